# Initial kernel scaffold; baseline (speedup 1.0000x reference)
import jax
import jax.numpy as jnp
from jax.experimental import pallas as pl


def kernel(left_x, left_graph_index, right_x, right_graph_index, left_x_batch, right_x_batch, left_diag_cnt, right_diag_cnt, diag_emb, diag_anc, diag_leaf, diag_Wl, diag_bl, diag_att, proc_emb, proc_anc, proc_leaf, proc_Wl, proc_bl, proc_att, atc_emb, atc_anc, atc_leaf, atc_Wl, atc_bl, atc_att, ntn_W, ntn_V, ntn_b, fc_W, fc_b):
    raise NotImplementedError("write your pallas kernel here")


# speedup vs baseline: 3.3561x; 3.3561x over previous
import functools

import jax
import jax.numpy as jnp
from jax import lax
from jax.experimental import pallas as pl
from jax.experimental.pallas import tpu as pltpu
from jax.experimental.pallas import tpu_sc as plsc

HID = 128
PAIR = 16
B = 4096
M = 200000
N_DIAG_LEAF, N_PROC_LEAF, N_ATC_LEAF = 10000, 4000, 4000
N_TOTAL_LEAF = N_DIAG_LEAF + N_PROC_LEAF + N_ATC_LEAF
USLOT = 16

NC, NS = 2, 16
NW = NC * NS

G_ROWS = 76800
G_PER_TILE = G_ROWS // NW
G_CHUNK = 480
G_NCHUNK = G_PER_TILE // G_CHUNK

B_PER_TILE = B // NW
T_CHUNK = 256
M_PAD = M + T_CHUNK


def _sc_mesh():
    return plsc.VectorSubcoreMesh(core_axis_name="c", subcore_axis_name="s")


def _leaf_gather_call(emb_all, anc_idx, leaf_idx):
    @functools.partial(
        pl.kernel,
        out_type=[
            jax.ShapeDtypeStruct((G_ROWS, HID), jnp.float32),
            jax.ShapeDtypeStruct((G_ROWS, HID), jnp.float32),
        ],
        mesh=_sc_mesh(),
        scratch_types=[
            pltpu.VMEM((G_CHUNK,), jnp.int32),
            pltpu.VMEM((G_CHUNK, HID), jnp.float32),
            pltpu.SemaphoreType.DMA,
        ],
    )
    def k(tab, aidx, lidx, ancg, leafg, idx_v, rows_v, sem):
        wid = lax.axis_index("s") * NC + lax.axis_index("c")
        base0 = wid * G_PER_TILE
        for idx_hbm, out_hbm in ((aidx, ancg), (lidx, leafg)):
            def chunk(i, carry, idx_hbm=idx_hbm, out_hbm=out_hbm):
                base = base0 + i * G_CHUNK
                pltpu.sync_copy(idx_hbm.at[pl.ds(base, G_CHUNK)], idx_v)
                pltpu.async_copy(tab.at[idx_v], rows_v, sem).wait()
                pltpu.sync_copy(rows_v, out_hbm.at[pl.ds(base, G_CHUNK)])
                return carry
            lax.fori_loop(0, G_NCHUNK, chunk, 0)

    return k(emb_all, anc_idx, leaf_idx)


def _leaf_mlp_kernel(anc_ref, leaf_ref, w1_ref, w2_ref, bl_ref, att_ref,
                     w_ref, s_ref, *, L, BL):
    i = pl.program_id(0)
    w1 = w1_ref[...]
    w2 = w2_ref[...]
    blv = bl_ref[...]
    att = att_ref[...]
    logits = []
    s_rows = []
    for l in range(L):
        anc_l = anc_ref[l]
        leaf_l = leaf_ref[l]
        h = jnp.tanh(
            jnp.dot(anc_l, w1, preferred_element_type=jnp.float32,
                    precision=lax.Precision.HIGHEST)
            + jnp.dot(leaf_l, w2, preferred_element_type=jnp.float32,
                      precision=lax.Precision.HIGHEST)
            + blv)
        logits.append(jnp.dot(h, att, preferred_element_type=jnp.float32,
                              precision=lax.Precision.HIGHEST))
        s_rows.append(jnp.sum(anc_l, axis=0, keepdims=True))
    logit = jnp.concatenate(logits, axis=1)
    m = jnp.max(logit, axis=1, keepdims=True)
    e = jnp.exp(logit - m)
    w_ref[...] = e / jnp.sum(e, axis=1, keepdims=True)
    s_rows.extend([jnp.zeros((1, HID), jnp.float32)] * (8 - L))
    s_pad = jnp.concatenate(s_rows, axis=0)

    @pl.when(i == 0)
    def _():
        s_ref[...] = jnp.zeros_like(s_ref)

    s_ref[...] += s_pad


def _leaf_mlp_call(anc3, leaf3, w1t, w2t, bl, att, *, L, nl):
    BL = 1000
    grid = nl // BL
    return pl.pallas_call(
        functools.partial(_leaf_mlp_kernel, L=L, BL=BL),
        grid=(grid,),
        in_specs=[
            pl.BlockSpec((L, BL, HID), lambda i: (0, i, 0)),
            pl.BlockSpec((L, BL, HID), lambda i: (0, i, 0)),
            pl.BlockSpec((HID, HID), lambda i: (0, 0)),
            pl.BlockSpec((HID, HID), lambda i: (0, 0)),
            pl.BlockSpec((1, HID), lambda i: (0, 0)),
            pl.BlockSpec((HID, 1), lambda i: (0, 0)),
        ],
        out_specs=[
            pl.BlockSpec((BL, L), lambda i: (i, 0)),
            pl.BlockSpec((8, HID), lambda i: (0, 0)),
        ],
        out_shape=[
            jax.ShapeDtypeStruct((nl, L), jnp.float32),
            jax.ShapeDtypeStruct((8, HID), jnp.float32),
        ],
    )(anc3, leaf3, w1t, w2t, bl, att)


def _token_pool_call(u, lx, lb, starts_l, rx, rb, starts_r):
    @functools.partial(
        pl.kernel,
        out_type=[
            jax.ShapeDtypeStruct((B, USLOT), jnp.float32),
            jax.ShapeDtypeStruct((B, USLOT), jnp.float32),
        ],
        mesh=_sc_mesh(),
        scratch_types=[
            pltpu.VMEM((48,), jnp.int32),
            pltpu.VMEM((T_CHUNK,), jnp.int32),
            pltpu.VMEM((T_CHUNK,), jnp.int32),
            pltpu.VMEM((T_CHUNK, USLOT), jnp.float32),
            pltpu.VMEM((B_PER_TILE, USLOT), jnp.float32),
            pltpu.SemaphoreType.DMA,
        ],
        compiler_params=pltpu.CompilerParams(use_tc_tiling_on_sc=False),
    )
    def k(u_hbm, lx_h, lb_h, sl_h, rx_h, rb_h, sr_h,
          ul_hbm, ur_hbm, st_v, idx_v, b_v, rows_v, acc, sem):
        wid = lax.axis_index("s") * NC + lax.axis_index("c")
        b_lo = wid * B_PER_TILE

        for x_h, bt_h, st_h, out_hbm in ((lx_h, lb_h, sl_h, ul_hbm),
                                         (rx_h, rb_h, sr_h, ur_hbm)):
            def zero(j, carry):
                acc[j] = jnp.zeros((USLOT,), jnp.float32)
                return carry
            lax.fori_loop(0, B_PER_TILE, zero, 0)

            pltpu.sync_copy(st_h, st_v)
            vals = []
            for part in range(3):
                vreg = st_v[pl.ds(part * 16, 16)]
                vals.extend(vreg[lane] for lane in range(16))
            t_lo = vals[0]
            t_hi = vals[1]
            for w in range(1, NW):
                t_lo = jnp.where(wid == w, vals[w], t_lo)
                t_hi = jnp.where(wid == w, vals[w + 1], t_hi)
            c_lo = (t_lo // T_CHUNK) * T_CHUNK
            n_ch = (t_hi - c_lo + T_CHUNK - 1) // T_CHUNK

            def chunk(ci, carry, x_h=x_h, bt_h=bt_h):
                base = c_lo + ci * T_CHUNK
                pltpu.sync_copy(x_h.at[pl.ds(base, T_CHUNK)], idx_v)
                pltpu.sync_copy(bt_h.at[pl.ds(base, T_CHUNK)], b_v)
                pltpu.async_copy(u_hbm.at[idx_v], rows_v, sem).wait()

                def grp(g, c2):
                    bvec = b_v[pl.ds(g * 16, 16)] - b_lo
                    for j in range(16):
                        bid = bvec[j]
                        r_i = g * 16 + j

                        @pl.when((bid >= 0) & (bid < B_PER_TILE))
                        def _(bid=bid, r_i=r_i):
                            plsc.addupdate(acc.at[bid], rows_v[r_i])
                    return c2
                lax.fori_loop(0, T_CHUNK // 16, grp, 0)
                return carry
            lax.fori_loop(0, n_ch, chunk, 0)
            pltpu.sync_copy(acc, out_hbm.at[pl.ds(b_lo, B_PER_TILE)])

    return k(u, lx, lb, starts_l, rx, rb, starts_r)


def _ntn_kernel(ul_ref, ur_ref, sb_ref, wr_ref, v1_ref, v2_ref, nb_ref,
                fw_ref, fb_ref, out_ref):
    hp = lax.Precision.HIGHEST
    sb = sb_ref[...]
    lg = jnp.dot(ul_ref[...], sb, preferred_element_type=jnp.float32,
                 precision=hp)
    rg = jnp.dot(ur_ref[...], sb, preferred_element_type=jnp.float32,
                 precision=hp)
    tmp = jnp.dot(lg, wr_ref[...], preferred_element_type=jnp.float32,
                  precision=hp)
    t1_cols = []
    for kk in range(PAIR):
        seg = tmp[:, kk * HID:(kk + 1) * HID] * rg
        t1_cols.append(jnp.sum(seg, axis=1, keepdims=True))
    t1 = jnp.concatenate(t1_cols, axis=1)
    t2 = (jnp.dot(lg, v1_ref[...], preferred_element_type=jnp.float32,
                  precision=hp)
          + jnp.dot(rg, v2_ref[...], preferred_element_type=jnp.float32,
                    precision=hp))
    pair = jax.nn.relu(t1 + t2 + nb_ref[...])
    o = jnp.dot(pair, fw_ref[...], preferred_element_type=jnp.float32,
                precision=hp) + fb_ref[...]
    out_ref[...] = jax.nn.sigmoid(o)


def _ntn_call(ul, ur, sbig, wr2, v1t, v2t, nb, fwt, fb):
    BB = 512
    grid = B // BB
    return pl.pallas_call(
        _ntn_kernel,
        grid=(grid,),
        in_specs=[
            pl.BlockSpec((BB, USLOT), lambda i: (i, 0)),
            pl.BlockSpec((BB, USLOT), lambda i: (i, 0)),
            pl.BlockSpec((USLOT, HID), lambda i: (0, 0)),
            pl.BlockSpec((HID, PAIR * HID), lambda i: (0, 0)),
            pl.BlockSpec((HID, PAIR), lambda i: (0, 0)),
            pl.BlockSpec((HID, PAIR), lambda i: (0, 0)),
            pl.BlockSpec((1, PAIR), lambda i: (0, 0)),
            pl.BlockSpec((PAIR, 1), lambda i: (0, 0)),
            pl.BlockSpec((1, 1), lambda i: (0, 0)),
        ],
        out_specs=pl.BlockSpec((BB, 1), lambda i: (i, 0)),
        out_shape=jax.ShapeDtypeStruct((B, 1), jnp.float32),
    )(ul, ur, sbig, wr2, v1t, v2t, nb, fwt, fb)


def kernel(left_x, left_graph_index, right_x, right_graph_index,
           left_x_batch, right_x_batch, left_diag_cnt, right_diag_cnt,
           diag_emb, diag_anc, diag_leaf, diag_Wl, diag_bl, diag_att,
           proc_emb, proc_anc, proc_leaf, proc_Wl, proc_bl, proc_att,
           atc_emb, atc_anc, atc_leaf, atc_Wl, atc_bl, atc_att,
           ntn_W, ntn_V, ntn_b, fc_W, fc_b):
    f32 = jnp.float32

    emb_all = jnp.concatenate([diag_emb, proc_emb, atc_emb], axis=0)
    offs = (0, diag_emb.shape[0], diag_emb.shape[0] + proc_emb.shape[0])
    anc_parts, leaf_parts = [], []
    for arr_a, arr_l, off in ((diag_anc, diag_leaf, offs[0]),
                              (proc_anc, proc_leaf, offs[1]),
                              (atc_anc, atc_leaf, offs[2])):
        anc_parts.append((arr_a + off).T.reshape(-1))
        leaf_parts.append((arr_l + off).T.reshape(-1))
    anc_idx = jnp.concatenate(anc_parts)
    leaf_idx = jnp.concatenate(leaf_parts)
    pad_n = G_ROWS - anc_idx.shape[0]
    anc_idx = jnp.concatenate([anc_idx, jnp.zeros((pad_n,), jnp.int32)])
    leaf_idx = jnp.concatenate([leaf_idx, jnp.zeros((pad_n,), jnp.int32)])

    ancg, leafg = _leaf_gather_call(emb_all, anc_idx, leaf_idx)

    sections = (
        (0, 4, N_DIAG_LEAF, diag_Wl, diag_bl, diag_att),
        (40000, 4, N_PROC_LEAF, proc_Wl, proc_bl, proc_att),
        (56000, 5, N_ATC_LEAF, atc_Wl, atc_bl, atc_att),
    )
    ws, ss = [], []
    for start, L, nl, Wl, bl, att in sections:
        anc3 = ancg[start:start + L * nl].reshape(L, nl, HID)
        leaf3 = leafg[start:start + L * nl].reshape(L, nl, HID)
        w1t = Wl[:, :HID].T
        w2t = Wl[:, HID:].T
        w_t, s_t = _leaf_mlp_call(anc3, leaf3, w1t, w2t,
                                  bl.reshape(1, HID), att, L=L, nl=nl)
        ws.append(w_t)
        ss.append(s_t)

    u = jnp.concatenate([
        jnp.pad(ws[0], ((0, 0), (0, USLOT - 4))),
        jnp.pad(ws[1], ((0, 0), (4, USLOT - 8))),
        jnp.pad(ws[2], ((0, 0), (8, USLOT - 13))),
    ], axis=0)
    sbig = jnp.concatenate([ss[0][:4], ss[1][:4], ss[2][:5],
                            jnp.zeros((3, HID), f32)], axis=0)

    lx = left_x.reshape(-1)
    rx = right_x.reshape(-1)
    lxp = jnp.concatenate([lx, jnp.zeros((M_PAD - M,), jnp.int32)])
    rxp = jnp.concatenate([rx, jnp.zeros((M_PAD - M,), jnp.int32)])
    lbp = jnp.concatenate([left_x_batch, jnp.full((M_PAD - M,), B, jnp.int32)])
    rbp = jnp.concatenate([right_x_batch, jnp.full((M_PAD - M,), B, jnp.int32)])
    edges = (jnp.arange(NW + 1, dtype=jnp.int32) * B_PER_TILE)
    starts_l = jnp.searchsorted(left_x_batch, edges).astype(jnp.int32)
    starts_r = jnp.searchsorted(right_x_batch, edges).astype(jnp.int32)
    starts_l = jnp.pad(starts_l, (0, 48 - NW - 1))
    starts_r = jnp.pad(starts_r, (0, 48 - NW - 1))

    ul, ur = _token_pool_call(u, lxp, lbp, starts_l, rxp, rbp, starts_r)

    wr2 = ntn_W.transpose(0, 2, 1).reshape(HID, PAIR * HID)
    out = _ntn_call(ul, ur, sbig,
                    wr2, ntn_V[:, :HID].T, ntn_V[:, HID:].T,
                    ntn_b.reshape(1, PAIR), fc_W.T, fc_b.reshape(1, 1))
    return out.reshape(-1)


# speedup vs baseline: 3.6422x; 1.0852x over previous
import functools

import jax
import jax.numpy as jnp
from jax import lax
from jax.experimental import pallas as pl
from jax.experimental.pallas import tpu as pltpu
from jax.experimental.pallas import tpu_sc as plsc

HID = 128
PAIR = 16
B = 4096
M = 200000
N_DIAG_LEAF, N_PROC_LEAF, N_ATC_LEAF = 10000, 4000, 4000
N_TOTAL_LEAF = N_DIAG_LEAF + N_PROC_LEAF + N_ATC_LEAF
USLOT = 16

NC, NS = 2, 16
NW = NC * NS

G_ROWS = 76800
G_PER_TILE = G_ROWS // NW
G_CHUNK = 400
G_NCHUNK = G_PER_TILE // G_CHUNK

B_PER_TILE = B // NW
T_CHUNK = 256
M_PAD = M + 2 * T_CHUNK


def _sc_mesh():
    return plsc.VectorSubcoreMesh(core_axis_name="c", subcore_axis_name="s")


def _leaf_gather_call(emb_all, anc_idx, leaf_idx):
    @functools.partial(
        pl.kernel,
        out_type=[
            jax.ShapeDtypeStruct((G_ROWS, HID), jnp.float32),
            jax.ShapeDtypeStruct((G_ROWS, HID), jnp.float32),
        ],
        mesh=_sc_mesh(),
        scratch_types=[
            pltpu.VMEM((G_PER_TILE,), jnp.int32),
            pltpu.VMEM((G_CHUNK, HID), jnp.float32),
            pltpu.VMEM((G_CHUNK, HID), jnp.float32),
            pltpu.SemaphoreType.DMA,
            pltpu.SemaphoreType.DMA,
            pltpu.SemaphoreType.DMA,
            pltpu.SemaphoreType.DMA,
        ],
    )
    def k(tab, aidx, lidx, ancg, leafg, idx_all, buf_a, buf_b,
          gs_a, gs_b, ws_a, ws_b):
        wid = lax.axis_index("s") * NC + lax.axis_index("c")
        base0 = wid * G_PER_TILE
        bufs = ((buf_a, gs_a, ws_a), (buf_b, gs_b, ws_b))
        for idx_hbm, out_hbm in ((aidx, ancg), (lidx, leafg)):
            pltpu.sync_copy(idx_hbm.at[pl.ds(base0, G_PER_TILE)], idx_all)

            def g_desc(i, buf, gsem):
                src = tab.at[idx_all.at[pl.ds(i * G_CHUNK, G_CHUNK)]]
                return pltpu.make_async_copy(src, buf, gsem)

            def w_desc(i, buf, wsem):
                dst = out_hbm.at[pl.ds(base0 + i * G_CHUNK, G_CHUNK)]
                return pltpu.make_async_copy(buf, dst, wsem)

            g_desc(0, buf_a, gs_a).start()
            g_desc(1, buf_b, gs_b).start()
            for i in range(G_NCHUNK):
                buf, gsem, wsem = bufs[i % 2]
                g_desc(i, buf, gsem).wait()
                w_desc(i, buf, wsem).start()
                if i + 2 < G_NCHUNK:
                    w_desc(i, buf, wsem).wait()
                    g_desc(i + 2, buf, gsem).start()
            for i in (G_NCHUNK - 2, G_NCHUNK - 1):
                buf, gsem, wsem = bufs[i % 2]
                w_desc(i, buf, wsem).wait()

    return k(emb_all, anc_idx, leaf_idx)


def _leaf_mlp_kernel(anc_ref, leaf_ref, w1_ref, w2_ref, bl_ref, att_ref,
                     w_ref, s_ref, *, L, BL):
    i = pl.program_id(0)
    w1 = w1_ref[...]
    w2 = w2_ref[...]
    blv = bl_ref[...]
    att = att_ref[...]
    logits = []
    s_rows = []
    for l in range(L):
        anc_l = anc_ref[l]
        leaf_l = leaf_ref[l]
        h = jnp.tanh(
            jnp.dot(anc_l, w1, preferred_element_type=jnp.float32,
                    precision=lax.Precision.HIGHEST)
            + jnp.dot(leaf_l, w2, preferred_element_type=jnp.float32,
                      precision=lax.Precision.HIGHEST)
            + blv)
        logits.append(jnp.dot(h, att, preferred_element_type=jnp.float32,
                              precision=lax.Precision.HIGHEST))
        s_rows.append(jnp.sum(anc_l, axis=0, keepdims=True))
    logit = jnp.concatenate(logits, axis=1)
    m = jnp.max(logit, axis=1, keepdims=True)
    e = jnp.exp(logit - m)
    w_ref[...] = e / jnp.sum(e, axis=1, keepdims=True)
    s_rows.extend([jnp.zeros((1, HID), jnp.float32)] * (8 - L))
    s_pad = jnp.concatenate(s_rows, axis=0)

    @pl.when(i == 0)
    def _():
        s_ref[...] = jnp.zeros_like(s_ref)

    s_ref[...] += s_pad


def _leaf_mlp_call(anc3, leaf3, w1t, w2t, bl, att, *, L, nl):
    BL = 1000
    grid = nl // BL
    return pl.pallas_call(
        functools.partial(_leaf_mlp_kernel, L=L, BL=BL),
        grid=(grid,),
        in_specs=[
            pl.BlockSpec((L, BL, HID), lambda i: (0, i, 0)),
            pl.BlockSpec((L, BL, HID), lambda i: (0, i, 0)),
            pl.BlockSpec((HID, HID), lambda i: (0, 0)),
            pl.BlockSpec((HID, HID), lambda i: (0, 0)),
            pl.BlockSpec((1, HID), lambda i: (0, 0)),
            pl.BlockSpec((HID, 1), lambda i: (0, 0)),
        ],
        out_specs=[
            pl.BlockSpec((BL, L), lambda i: (i, 0)),
            pl.BlockSpec((8, HID), lambda i: (0, 0)),
        ],
        out_shape=[
            jax.ShapeDtypeStruct((nl, L), jnp.float32),
            jax.ShapeDtypeStruct((8, HID), jnp.float32),
        ],
    )(anc3, leaf3, w1t, w2t, bl, att)


def _token_pool_call(u, lx, lb, starts_l, rx, rb, starts_r):
    @functools.partial(
        pl.kernel,
        out_type=[
            jax.ShapeDtypeStruct((B, USLOT), jnp.float32),
            jax.ShapeDtypeStruct((B, USLOT), jnp.float32),
        ],
        mesh=_sc_mesh(),
        scratch_types=[
            pltpu.VMEM((48,), jnp.int32),
            pltpu.VMEM((T_CHUNK,), jnp.int32),
            pltpu.VMEM((T_CHUNK,), jnp.int32),
            pltpu.VMEM((T_CHUNK,), jnp.int32),
            pltpu.VMEM((T_CHUNK,), jnp.int32),
            pltpu.VMEM((T_CHUNK, USLOT), jnp.float32),
            pltpu.VMEM((T_CHUNK, USLOT), jnp.float32),
            pltpu.VMEM((B_PER_TILE, USLOT), jnp.float32),
            pltpu.SemaphoreType.DMA,
            pltpu.SemaphoreType.DMA,
        ],
        compiler_params=pltpu.CompilerParams(use_tc_tiling_on_sc=False),
    )
    def k(u_hbm, lx_h, lb_h, sl_h, rx_h, rb_h, sr_h,
          ul_hbm, ur_hbm, st_v, idx_a, idx_b, bv_a, bv_b,
          rows_a, rows_b, acc, gs_a, gs_b):
        wid = lax.axis_index("s") * NC + lax.axis_index("c")
        b_lo = wid * B_PER_TILE

        for x_h, bt_h, st_h, out_hbm in ((lx_h, lb_h, sl_h, ul_hbm),
                                         (rx_h, rb_h, sr_h, ur_hbm)):
            def zero(j, carry):
                acc[j] = jnp.zeros((USLOT,), jnp.float32)
                return carry
            lax.fori_loop(0, B_PER_TILE, zero, 0)

            pltpu.sync_copy(st_h, st_v)
            vals = []
            for part in range(3):
                vreg = st_v[pl.ds(part * 16, 16)]
                vals.extend(vreg[lane] for lane in range(16))
            t_lo = vals[0]
            t_hi = vals[1]
            for w in range(1, NW):
                t_lo = jnp.where(wid == w, vals[w], t_lo)
                t_hi = jnp.where(wid == w, vals[w + 1], t_hi)
            c_lo = (t_lo // T_CHUNK) * T_CHUNK
            n_ch = (t_hi - c_lo + T_CHUNK - 1) // T_CHUNK
            n_pair = jnp.maximum((n_ch + 1) // 2, 1)
            pipe = ((idx_a, bv_a, rows_a, gs_a), (idx_b, bv_b, rows_b, gs_b))

            def fetch(base, idx_v, bv_v, rows_v, gsem, x_h=x_h, bt_h=bt_h):
                pltpu.sync_copy(x_h.at[pl.ds(base, T_CHUNK)], idx_v)
                pltpu.sync_copy(bt_h.at[pl.ds(base, T_CHUNK)], bv_v)
                pltpu.make_async_copy(u_hbm.at[idx_v], rows_v, gsem).start()

            for b in range(2):
                fetch(c_lo + b * T_CHUNK, *pipe[b])

            def pair(p, carry):
                for b in range(2):
                    idx_v, bv_v, rows_v, gsem = pipe[b]
                    base = c_lo + (2 * p + b) * T_CHUNK
                    pltpu.make_async_copy(u_hbm.at[idx_v], rows_v,
                                          gsem).wait()

                    def grp(g, c2, bv_v=bv_v, rows_v=rows_v):
                        bvec = bv_v[pl.ds(g * 16, 16)] - b_lo
                        for j in range(16):
                            bid = bvec[j]
                            r_i = g * 16 + j

                            @pl.when((bid >= 0) & (bid < B_PER_TILE))
                            def _(bid=bid, r_i=r_i):
                                plsc.addupdate(acc.at[bid], rows_v[r_i])
                        return c2
                    lax.fori_loop(0, T_CHUNK // 16, grp, 0)

                    @pl.when(p < n_pair - 1)
                    def _(base=base, idx_v=idx_v, bv_v=bv_v,
                          rows_v=rows_v, gsem=gsem):
                        fetch(base + 2 * T_CHUNK, idx_v, bv_v, rows_v, gsem)
                return carry
            lax.fori_loop(0, n_pair, pair, 0)
            pltpu.sync_copy(acc, out_hbm.at[pl.ds(b_lo, B_PER_TILE)])

    return k(u, lx, lb, starts_l, rx, rb, starts_r)


def _ntn_kernel(ul_ref, ur_ref, sb_ref, wr_ref, v1_ref, v2_ref, nb_ref,
                fw_ref, fb_ref, out_ref):
    hp = lax.Precision.HIGHEST
    sb = sb_ref[...]
    lg = jnp.dot(ul_ref[...], sb, preferred_element_type=jnp.float32,
                 precision=hp)
    rg = jnp.dot(ur_ref[...], sb, preferred_element_type=jnp.float32,
                 precision=hp)
    tmp = jnp.dot(lg, wr_ref[...], preferred_element_type=jnp.float32,
                  precision=hp)
    t1_cols = []
    for kk in range(PAIR):
        seg = tmp[:, kk * HID:(kk + 1) * HID] * rg
        t1_cols.append(jnp.sum(seg, axis=1, keepdims=True))
    t1 = jnp.concatenate(t1_cols, axis=1)
    t2 = (jnp.dot(lg, v1_ref[...], preferred_element_type=jnp.float32,
                  precision=hp)
          + jnp.dot(rg, v2_ref[...], preferred_element_type=jnp.float32,
                    precision=hp))
    pair = jax.nn.relu(t1 + t2 + nb_ref[...])
    o = jnp.dot(pair, fw_ref[...], preferred_element_type=jnp.float32,
                precision=hp) + fb_ref[...]
    out_ref[...] = jax.nn.sigmoid(o)


def _ntn_call(ul, ur, sbig, wr2, v1t, v2t, nb, fwt, fb):
    BB = 512
    grid = B // BB
    return pl.pallas_call(
        _ntn_kernel,
        grid=(grid,),
        in_specs=[
            pl.BlockSpec((BB, USLOT), lambda i: (i, 0)),
            pl.BlockSpec((BB, USLOT), lambda i: (i, 0)),
            pl.BlockSpec((USLOT, HID), lambda i: (0, 0)),
            pl.BlockSpec((HID, PAIR * HID), lambda i: (0, 0)),
            pl.BlockSpec((HID, PAIR), lambda i: (0, 0)),
            pl.BlockSpec((HID, PAIR), lambda i: (0, 0)),
            pl.BlockSpec((1, PAIR), lambda i: (0, 0)),
            pl.BlockSpec((PAIR, 1), lambda i: (0, 0)),
            pl.BlockSpec((1, 1), lambda i: (0, 0)),
        ],
        out_specs=pl.BlockSpec((BB, 1), lambda i: (i, 0)),
        out_shape=jax.ShapeDtypeStruct((B, 1), jnp.float32),
    )(ul, ur, sbig, wr2, v1t, v2t, nb, fwt, fb)


def kernel(left_x, left_graph_index, right_x, right_graph_index,
           left_x_batch, right_x_batch, left_diag_cnt, right_diag_cnt,
           diag_emb, diag_anc, diag_leaf, diag_Wl, diag_bl, diag_att,
           proc_emb, proc_anc, proc_leaf, proc_Wl, proc_bl, proc_att,
           atc_emb, atc_anc, atc_leaf, atc_Wl, atc_bl, atc_att,
           ntn_W, ntn_V, ntn_b, fc_W, fc_b):
    f32 = jnp.float32

    emb_all = jnp.concatenate([diag_emb, proc_emb, atc_emb], axis=0)
    offs = (0, diag_emb.shape[0], diag_emb.shape[0] + proc_emb.shape[0])
    anc_parts, leaf_parts = [], []
    for arr_a, arr_l, off in ((diag_anc, diag_leaf, offs[0]),
                              (proc_anc, proc_leaf, offs[1]),
                              (atc_anc, atc_leaf, offs[2])):
        anc_parts.append((arr_a + off).T.reshape(-1))
        leaf_parts.append((arr_l + off).T.reshape(-1))
    anc_idx = jnp.concatenate(anc_parts)
    leaf_idx = jnp.concatenate(leaf_parts)
    pad_n = G_ROWS - anc_idx.shape[0]
    anc_idx = jnp.concatenate([anc_idx, jnp.zeros((pad_n,), jnp.int32)])
    leaf_idx = jnp.concatenate([leaf_idx, jnp.zeros((pad_n,), jnp.int32)])

    ancg, leafg = _leaf_gather_call(emb_all, anc_idx, leaf_idx)

    sections = (
        (0, 4, N_DIAG_LEAF, diag_Wl, diag_bl, diag_att),
        (40000, 4, N_PROC_LEAF, proc_Wl, proc_bl, proc_att),
        (56000, 5, N_ATC_LEAF, atc_Wl, atc_bl, atc_att),
    )
    ws, ss = [], []
    for start, L, nl, Wl, bl, att in sections:
        anc3 = ancg[start:start + L * nl].reshape(L, nl, HID)
        leaf3 = leafg[start:start + L * nl].reshape(L, nl, HID)
        w1t = Wl[:, :HID].T
        w2t = Wl[:, HID:].T
        w_t, s_t = _leaf_mlp_call(anc3, leaf3, w1t, w2t,
                                  bl.reshape(1, HID), att, L=L, nl=nl)
        ws.append(w_t)
        ss.append(s_t)

    u = jnp.concatenate([
        jnp.pad(ws[0], ((0, 0), (0, USLOT - 4))),
        jnp.pad(ws[1], ((0, 0), (4, USLOT - 8))),
        jnp.pad(ws[2], ((0, 0), (8, USLOT - 13))),
    ], axis=0)
    sbig = jnp.concatenate([ss[0][:4], ss[1][:4], ss[2][:5],
                            jnp.zeros((3, HID), f32)], axis=0)

    lx = left_x.reshape(-1)
    rx = right_x.reshape(-1)
    lxp = jnp.concatenate([lx, jnp.zeros((M_PAD - M,), jnp.int32)])
    rxp = jnp.concatenate([rx, jnp.zeros((M_PAD - M,), jnp.int32)])
    lbp = jnp.concatenate([left_x_batch, jnp.full((M_PAD - M,), B, jnp.int32)])
    rbp = jnp.concatenate([right_x_batch, jnp.full((M_PAD - M,), B, jnp.int32)])
    edges = (jnp.arange(NW + 1, dtype=jnp.int32) * B_PER_TILE)
    starts_l = jnp.searchsorted(left_x_batch, edges).astype(jnp.int32)
    starts_r = jnp.searchsorted(right_x_batch, edges).astype(jnp.int32)
    starts_l = jnp.pad(starts_l, (0, 48 - NW - 1))
    starts_r = jnp.pad(starts_r, (0, 48 - NW - 1))

    ul, ur = _token_pool_call(u, lxp, lbp, starts_l, rxp, rbp, starts_r)

    wr2 = ntn_W.transpose(0, 2, 1).reshape(HID, PAIR * HID)
    out = _ntn_call(ul, ur, sbig,
                    wr2, ntn_V[:, :HID].T, ntn_V[:, HID:].T,
                    ntn_b.reshape(1, PAIR), fc_W.T, fc_b.reshape(1, 1))
    return out.reshape(-1)


# speedup vs baseline: 3.7433x; 1.0277x over previous
import functools

import jax
import jax.numpy as jnp
from jax import lax
from jax.experimental import pallas as pl
from jax.experimental.pallas import tpu as pltpu
from jax.experimental.pallas import tpu_sc as plsc

HID = 128
PAIR = 16
B = 4096
M = 200000
N_DIAG_LEAF, N_PROC_LEAF, N_ATC_LEAF = 10000, 4000, 4000
N_TOTAL_LEAF = N_DIAG_LEAF + N_PROC_LEAF + N_ATC_LEAF
USLOT = 16

NC, NS = 2, 16
NW = NC * NS

G_ROWS = 76800
G_PER_TILE = G_ROWS // NW
G_CHUNK = 400
G_NCHUNK = G_PER_TILE // G_CHUNK

B_PER_TILE = B // NW
T_CHUNK = 512
M_PAD = M + 2 * T_CHUNK
ACC_ROWS = B_PER_TILE + 8


def _sc_mesh():
    return plsc.VectorSubcoreMesh(core_axis_name="c", subcore_axis_name="s")


def _leaf_gather_call(emb_all, anc_idx, leaf_idx):
    @functools.partial(
        pl.kernel,
        out_type=[
            jax.ShapeDtypeStruct((G_ROWS, HID), jnp.float32),
            jax.ShapeDtypeStruct((G_ROWS, HID), jnp.float32),
        ],
        mesh=_sc_mesh(),
        scratch_types=[
            pltpu.VMEM((G_PER_TILE,), jnp.int32),
            pltpu.VMEM((G_CHUNK, HID), jnp.float32),
            pltpu.VMEM((G_CHUNK, HID), jnp.float32),
            pltpu.SemaphoreType.DMA,
            pltpu.SemaphoreType.DMA,
            pltpu.SemaphoreType.DMA,
            pltpu.SemaphoreType.DMA,
        ],
    )
    def k(tab, aidx, lidx, ancg, leafg, idx_all, buf_a, buf_b,
          gs_a, gs_b, ws_a, ws_b):
        wid = lax.axis_index("s") * NC + lax.axis_index("c")
        base0 = wid * G_PER_TILE
        bufs = ((buf_a, gs_a, ws_a), (buf_b, gs_b, ws_b))
        for idx_hbm, out_hbm in ((aidx, ancg), (lidx, leafg)):
            pltpu.sync_copy(idx_hbm.at[pl.ds(base0, G_PER_TILE)], idx_all)

            def g_desc(i, buf, gsem):
                src = tab.at[idx_all.at[pl.ds(i * G_CHUNK, G_CHUNK)]]
                return pltpu.make_async_copy(src, buf, gsem)

            def w_desc(i, buf, wsem):
                dst = out_hbm.at[pl.ds(base0 + i * G_CHUNK, G_CHUNK)]
                return pltpu.make_async_copy(buf, dst, wsem)

            g_desc(0, buf_a, gs_a).start()
            g_desc(1, buf_b, gs_b).start()
            for i in range(G_NCHUNK):
                buf, gsem, wsem = bufs[i % 2]
                g_desc(i, buf, gsem).wait()
                w_desc(i, buf, wsem).start()
                if i + 2 < G_NCHUNK:
                    w_desc(i, buf, wsem).wait()
                    g_desc(i + 2, buf, gsem).start()
            for i in (G_NCHUNK - 2, G_NCHUNK - 1):
                buf, gsem, wsem = bufs[i % 2]
                w_desc(i, buf, wsem).wait()

    return k(emb_all, anc_idx, leaf_idx)


def _leaf_mlp_kernel(anc_ref, leaf_ref, w1_ref, w2_ref, bl_ref, att_ref,
                     w_ref, s_ref, *, L, BL):
    i = pl.program_id(0)
    w1 = w1_ref[...]
    w2 = w2_ref[...]
    blv = bl_ref[...]
    att = att_ref[...]
    logits = []
    s_rows = []
    for l in range(L):
        anc_l = anc_ref[l]
        leaf_l = leaf_ref[l]
        h = jnp.tanh(
            jnp.dot(anc_l, w1, preferred_element_type=jnp.float32,
                    precision=lax.Precision.HIGHEST)
            + jnp.dot(leaf_l, w2, preferred_element_type=jnp.float32,
                      precision=lax.Precision.HIGHEST)
            + blv)
        logits.append(jnp.dot(h, att, preferred_element_type=jnp.float32,
                              precision=lax.Precision.HIGHEST))
        s_rows.append(jnp.sum(anc_l, axis=0, keepdims=True))
    logit = jnp.concatenate(logits, axis=1)
    m = jnp.max(logit, axis=1, keepdims=True)
    e = jnp.exp(logit - m)
    w_ref[...] = e / jnp.sum(e, axis=1, keepdims=True)
    s_rows.extend([jnp.zeros((1, HID), jnp.float32)] * (8 - L))
    s_pad = jnp.concatenate(s_rows, axis=0)

    @pl.when(i == 0)
    def _():
        s_ref[...] = jnp.zeros_like(s_ref)

    s_ref[...] += s_pad


def _leaf_mlp_call(anc3, leaf3, w1t, w2t, bl, att, *, L, nl):
    BL = 1000
    grid = nl // BL
    return pl.pallas_call(
        functools.partial(_leaf_mlp_kernel, L=L, BL=BL),
        grid=(grid,),
        in_specs=[
            pl.BlockSpec((L, BL, HID), lambda i: (0, i, 0)),
            pl.BlockSpec((L, BL, HID), lambda i: (0, i, 0)),
            pl.BlockSpec((HID, HID), lambda i: (0, 0)),
            pl.BlockSpec((HID, HID), lambda i: (0, 0)),
            pl.BlockSpec((1, HID), lambda i: (0, 0)),
            pl.BlockSpec((HID, 1), lambda i: (0, 0)),
        ],
        out_specs=[
            pl.BlockSpec((BL, L), lambda i: (i, 0)),
            pl.BlockSpec((8, HID), lambda i: (0, 0)),
        ],
        out_shape=[
            jax.ShapeDtypeStruct((nl, L), jnp.float32),
            jax.ShapeDtypeStruct((8, HID), jnp.float32),
        ],
    )(anc3, leaf3, w1t, w2t, bl, att)


def _token_pool_call(u, lx, lb, starts_l, rx, rb, starts_r):
    @functools.partial(
        pl.kernel,
        out_type=[
            jax.ShapeDtypeStruct((B, USLOT), jnp.float32),
            jax.ShapeDtypeStruct((B, USLOT), jnp.float32),
        ],
        mesh=_sc_mesh(),
        scratch_types=[
            pltpu.VMEM((48,), jnp.int32),
            pltpu.VMEM((T_CHUNK,), jnp.int32),
            pltpu.VMEM((T_CHUNK,), jnp.int32),
            pltpu.VMEM((T_CHUNK,), jnp.int32),
            pltpu.VMEM((T_CHUNK,), jnp.int32),
            pltpu.VMEM((T_CHUNK, USLOT), jnp.float32),
            pltpu.VMEM((T_CHUNK, USLOT), jnp.float32),
            pltpu.VMEM((ACC_ROWS, USLOT), jnp.float32),
            pltpu.SemaphoreType.DMA,
            pltpu.SemaphoreType.DMA,
        ],
        compiler_params=pltpu.CompilerParams(use_tc_tiling_on_sc=False),
    )
    def k(u_hbm, lx_h, lb_h, sl_h, rx_h, rb_h, sr_h,
          ul_hbm, ur_hbm, st_v, idx_a, idx_b, bv_a, bv_b,
          rows_a, rows_b, acc, gs_a, gs_b):
        wid = lax.axis_index("s") * NC + lax.axis_index("c")
        b_lo = wid * B_PER_TILE

        for x_h, bt_h, st_h, out_hbm in ((lx_h, lb_h, sl_h, ul_hbm),
                                         (rx_h, rb_h, sr_h, ur_hbm)):
            def zero(j, carry):
                acc[j] = jnp.zeros((USLOT,), jnp.float32)
                return carry
            lax.fori_loop(0, ACC_ROWS, zero, 0)

            pltpu.sync_copy(st_h, st_v)
            vals = []
            for part in range(3):
                vreg = st_v[pl.ds(part * 16, 16)]
                vals.extend(vreg[lane] for lane in range(16))
            t_lo = vals[0]
            t_hi = vals[1]
            for w in range(1, NW):
                t_lo = jnp.where(wid == w, vals[w], t_lo)
                t_hi = jnp.where(wid == w, vals[w + 1], t_hi)
            c_lo = (t_lo // T_CHUNK) * T_CHUNK
            n_ch = (t_hi - c_lo + T_CHUNK - 1) // T_CHUNK
            n_pair = jnp.maximum((n_ch + 1) // 2, 1)
            pipe = ((idx_a, bv_a, rows_a, gs_a), (idx_b, bv_b, rows_b, gs_b))

            def fetch(base, idx_v, bv_v, rows_v, gsem, x_h=x_h, bt_h=bt_h):
                pltpu.sync_copy(x_h.at[pl.ds(base, T_CHUNK)], idx_v)
                pltpu.sync_copy(bt_h.at[pl.ds(base, T_CHUNK)], bv_v)
                pltpu.make_async_copy(u_hbm.at[idx_v], rows_v, gsem).start()

            for b in range(2):
                fetch(c_lo + b * T_CHUNK, *pipe[b])

            def pair(p, carry):
                for b in range(2):
                    idx_v, bv_v, rows_v, gsem = pipe[b]
                    base = c_lo + (2 * p + b) * T_CHUNK
                    pltpu.make_async_copy(u_hbm.at[idx_v], rows_v,
                                          gsem).wait()

                    def grp(g, c2, bv_v=bv_v, rows_v=rows_v):
                        bvec = bv_v[pl.ds(g * 16, 16)] - b_lo
                        ok = (bvec >= 0) & (bvec < B_PER_TILE)
                        tgt = jnp.where(ok, bvec, B_PER_TILE)
                        for j in range(16):
                            plsc.addupdate(acc.at[tgt[j]],
                                           rows_v[g * 16 + j])
                        return c2
                    lax.fori_loop(0, T_CHUNK // 16, grp, 0)

                    @pl.when(p < n_pair - 1)
                    def _(base=base, idx_v=idx_v, bv_v=bv_v,
                          rows_v=rows_v, gsem=gsem):
                        fetch(base + 2 * T_CHUNK, idx_v, bv_v, rows_v, gsem)
                return carry
            lax.fori_loop(0, n_pair, pair, 0)
            pltpu.sync_copy(acc.at[pl.ds(0, B_PER_TILE)],
                            out_hbm.at[pl.ds(b_lo, B_PER_TILE)])

    return k(u, lx, lb, starts_l, rx, rb, starts_r)


def _ntn_kernel(ul_ref, ur_ref, sb_ref, wr_ref, v1_ref, v2_ref, nb_ref,
                fw_ref, fb_ref, out_ref):
    hp = lax.Precision.HIGHEST
    sb = sb_ref[...]
    lg = jnp.dot(ul_ref[...], sb, preferred_element_type=jnp.float32,
                 precision=hp)
    rg = jnp.dot(ur_ref[...], sb, preferred_element_type=jnp.float32,
                 precision=hp)
    tmp = jnp.dot(lg, wr_ref[...], preferred_element_type=jnp.float32,
                  precision=hp)
    t1_cols = []
    for kk in range(PAIR):
        seg = tmp[:, kk * HID:(kk + 1) * HID] * rg
        t1_cols.append(jnp.sum(seg, axis=1, keepdims=True))
    t1 = jnp.concatenate(t1_cols, axis=1)
    t2 = (jnp.dot(lg, v1_ref[...], preferred_element_type=jnp.float32,
                  precision=hp)
          + jnp.dot(rg, v2_ref[...], preferred_element_type=jnp.float32,
                    precision=hp))
    pair = jax.nn.relu(t1 + t2 + nb_ref[...])
    o = jnp.dot(pair, fw_ref[...], preferred_element_type=jnp.float32,
                precision=hp) + fb_ref[...]
    out_ref[...] = jax.nn.sigmoid(o)


def _ntn_call(ul, ur, sbig, wr2, v1t, v2t, nb, fwt, fb):
    BB = 512
    grid = B // BB
    return pl.pallas_call(
        _ntn_kernel,
        grid=(grid,),
        in_specs=[
            pl.BlockSpec((BB, USLOT), lambda i: (i, 0)),
            pl.BlockSpec((BB, USLOT), lambda i: (i, 0)),
            pl.BlockSpec((USLOT, HID), lambda i: (0, 0)),
            pl.BlockSpec((HID, PAIR * HID), lambda i: (0, 0)),
            pl.BlockSpec((HID, PAIR), lambda i: (0, 0)),
            pl.BlockSpec((HID, PAIR), lambda i: (0, 0)),
            pl.BlockSpec((1, PAIR), lambda i: (0, 0)),
            pl.BlockSpec((PAIR, 1), lambda i: (0, 0)),
            pl.BlockSpec((1, 1), lambda i: (0, 0)),
        ],
        out_specs=pl.BlockSpec((BB, 1), lambda i: (i, 0)),
        out_shape=jax.ShapeDtypeStruct((B, 1), jnp.float32),
    )(ul, ur, sbig, wr2, v1t, v2t, nb, fwt, fb)


def kernel(left_x, left_graph_index, right_x, right_graph_index,
           left_x_batch, right_x_batch, left_diag_cnt, right_diag_cnt,
           diag_emb, diag_anc, diag_leaf, diag_Wl, diag_bl, diag_att,
           proc_emb, proc_anc, proc_leaf, proc_Wl, proc_bl, proc_att,
           atc_emb, atc_anc, atc_leaf, atc_Wl, atc_bl, atc_att,
           ntn_W, ntn_V, ntn_b, fc_W, fc_b):
    f32 = jnp.float32

    emb_all = jnp.concatenate([diag_emb, proc_emb, atc_emb], axis=0)
    offs = (0, diag_emb.shape[0], diag_emb.shape[0] + proc_emb.shape[0])
    anc_parts, leaf_parts = [], []
    for arr_a, arr_l, off in ((diag_anc, diag_leaf, offs[0]),
                              (proc_anc, proc_leaf, offs[1]),
                              (atc_anc, atc_leaf, offs[2])):
        anc_parts.append((arr_a + off).T.reshape(-1))
        leaf_parts.append((arr_l + off).T.reshape(-1))
    anc_idx = jnp.concatenate(anc_parts)
    leaf_idx = jnp.concatenate(leaf_parts)
    pad_n = G_ROWS - anc_idx.shape[0]
    anc_idx = jnp.concatenate([anc_idx, jnp.zeros((pad_n,), jnp.int32)])
    leaf_idx = jnp.concatenate([leaf_idx, jnp.zeros((pad_n,), jnp.int32)])

    ancg, leafg = _leaf_gather_call(emb_all, anc_idx, leaf_idx)

    sections = (
        (0, 4, N_DIAG_LEAF, diag_Wl, diag_bl, diag_att),
        (40000, 4, N_PROC_LEAF, proc_Wl, proc_bl, proc_att),
        (56000, 5, N_ATC_LEAF, atc_Wl, atc_bl, atc_att),
    )
    ws, ss = [], []
    for start, L, nl, Wl, bl, att in sections:
        anc3 = ancg[start:start + L * nl].reshape(L, nl, HID)
        leaf3 = leafg[start:start + L * nl].reshape(L, nl, HID)
        w1t = Wl[:, :HID].T
        w2t = Wl[:, HID:].T
        w_t, s_t = _leaf_mlp_call(anc3, leaf3, w1t, w2t,
                                  bl.reshape(1, HID), att, L=L, nl=nl)
        ws.append(w_t)
        ss.append(s_t)

    u = jnp.concatenate([
        jnp.pad(ws[0], ((0, 0), (0, USLOT - 4))),
        jnp.pad(ws[1], ((0, 0), (4, USLOT - 8))),
        jnp.pad(ws[2], ((0, 0), (8, USLOT - 13))),
    ], axis=0)
    sbig = jnp.concatenate([ss[0][:4], ss[1][:4], ss[2][:5],
                            jnp.zeros((3, HID), f32)], axis=0)

    lx = left_x.reshape(-1)
    rx = right_x.reshape(-1)
    lxp = jnp.concatenate([lx, jnp.zeros((M_PAD - M,), jnp.int32)])
    rxp = jnp.concatenate([rx, jnp.zeros((M_PAD - M,), jnp.int32)])
    lbp = jnp.concatenate([left_x_batch, jnp.full((M_PAD - M,), B, jnp.int32)])
    rbp = jnp.concatenate([right_x_batch, jnp.full((M_PAD - M,), B, jnp.int32)])
    edges = (jnp.arange(NW + 1, dtype=jnp.int32) * B_PER_TILE)
    starts_l = jnp.searchsorted(left_x_batch, edges).astype(jnp.int32)
    starts_r = jnp.searchsorted(right_x_batch, edges).astype(jnp.int32)
    starts_l = jnp.pad(starts_l, (0, 48 - NW - 1))
    starts_r = jnp.pad(starts_r, (0, 48 - NW - 1))

    ul, ur = _token_pool_call(u, lxp, lbp, starts_l, rxp, rbp, starts_r)

    wr2 = ntn_W.transpose(0, 2, 1).reshape(HID, PAIR * HID)
    out = _ntn_call(ul, ur, sbig,
                    wr2, ntn_V[:, :HID].T, ntn_V[:, HID:].T,
                    ntn_b.reshape(1, PAIR), fc_W.T, fc_b.reshape(1, 1))
    return out.reshape(-1)
